# baseline (speedup 1.0000x reference)
import functools

import jax
import jax.numpy as jnp
from jax import lax
from jax.experimental import pallas as pl
from jax.experimental.pallas import tpu as pltpu
from jax.experimental.pallas import tpu_sc as plsc

N_ROWS = 100000
EMB = 64
B = 16384

_NC = 2
_NS = 16
_NW = _NC * _NS
_BW = B // _NW
_L = 16
_C = 512


def _mf_body(user_hbm, item_hbm, uemb_hbm, iemb_hbm, ubias_hbm, ibias_hbm,
             out_hbm, uidx_v, iidx_v, ue_v, ie_v, ub_v, ib_v, o_v, tmp_v,
             usem, isem, ubsem, ibsem):
    wid = lax.axis_index("s") * _NC + lax.axis_index("c")
    base = wid * _BW

    pltpu.sync_copy(user_hbm.at[pl.ds(base, _BW)], uidx_v)
    pltpu.sync_copy(item_hbm.at[pl.ds(base, _BW)], iidx_v)

    lanes = lax.iota(jnp.int32, _L)

    bias_cps = [
        pltpu.async_copy(ubias_hbm.at[uidx_v], ub_v, ubsem),
        pltpu.async_copy(ibias_hbm.at[iidx_v], ib_v, ibsem),
    ]
    for cp in bias_cps:
        cp.wait()

    def chunk(p, carry):
        pbase = p * _C

        emb_cps = [
            pltpu.async_copy(uemb_hbm.at[uidx_v.at[pl.ds(pbase, _C)]], ue_v, usem),
            pltpu.async_copy(iemb_hbm.at[iidx_v.at[pl.ds(pbase, _C)]], ie_v, isem),
        ]
        for cp in emb_cps:
            cp.wait()

        def group(g, carry2):
            for r in range(_L):
                row = g * _L + r
                acc = ue_v[row, pl.ds(0, _L)] * ie_v[row, pl.ds(0, _L)]
                for c in range(1, EMB // _L):
                    acc = acc + (ue_v[row, pl.ds(c * _L, _L)]
                                 * ie_v[row, pl.ds(c * _L, _L)])
                plsc.store_scatter(tmp_v, [lanes * _L + r], acc)
            tot = tmp_v[pl.ds(0, _L)]
            for j in range(1, _L):
                tot = tot + tmp_v[pl.ds(j * _L, _L)]
            osl = pl.ds(pbase + g * _L, _L)
            o_v[osl] = tot + ub_v[osl] + ib_v[osl]
            return carry2

        lax.fori_loop(0, _C // _L, group, 0)
        return carry

    lax.fori_loop(0, _BW // _C, chunk, 0)

    pltpu.sync_copy(o_v, out_hbm.at[pl.ds(base, _BW)])


@jax.jit
def _mf_call(user, item, uemb, iemb, ubias, ibias):
    mesh = plsc.VectorSubcoreMesh(core_axis_name="c", subcore_axis_name="s")
    f = functools.partial(
        pl.kernel,
        out_type=jax.ShapeDtypeStruct((B,), jnp.float32),
        mesh=mesh,
        compiler_params=pltpu.CompilerParams(
            needs_layout_passes=False,
            use_tc_tiling_on_sc=False,
        ),
        scratch_types=[
            pltpu.VMEM((_BW,), jnp.int32),
            pltpu.VMEM((_BW,), jnp.int32),
            pltpu.VMEM((_C, EMB), jnp.float32),
            pltpu.VMEM((_C, EMB), jnp.float32),
            pltpu.VMEM((_BW,), jnp.float32),
            pltpu.VMEM((_BW,), jnp.float32),
            pltpu.VMEM((_BW,), jnp.float32),
            pltpu.VMEM((_L * _L,), jnp.float32),
            pltpu.SemaphoreType.DMA,
            pltpu.SemaphoreType.DMA,
            pltpu.SemaphoreType.DMA,
            pltpu.SemaphoreType.DMA,
        ],
    )(_mf_body)
    return f(user, item, uemb, iemb, ubias, ibias)


def kernel(user, item, user_emb_w, item_emb_w, user_bias_w, item_bias_w):
    return _mf_call(
        user.astype(jnp.int32),
        item.astype(jnp.int32),
        user_emb_w,
        item_emb_w,
        user_bias_w.reshape(-1),
        item_bias_w.reshape(-1),
    )


# speedup vs baseline: 1.3201x; 1.3201x over previous
import functools

import jax
import jax.numpy as jnp
from jax import lax
from jax.experimental import pallas as pl
from jax.experimental.pallas import tpu as pltpu
from jax.experimental.pallas import tpu_sc as plsc

N_ROWS = 100000
EMB = 64
B = 16384

_NC = 2
_NS = 16
_NW = _NC * _NS
_BW = B // _NW
_L = 16
_C = 128
_NB = 2


def _mf_body(user_hbm, item_hbm, uemb_hbm, iemb_hbm, ubias_hbm, ibias_hbm,
             out_hbm, uidx_v, iidx_v, ue_v, ie_v, ub_v, ib_v, o_v, tmp_v,
             usem, isem, ubsem, ibsem):
    wid = lax.axis_index("s") * _NC + lax.axis_index("c")
    base = wid * _BW

    pltpu.sync_copy(user_hbm.at[pl.ds(base, _BW)], uidx_v)
    pltpu.sync_copy(item_hbm.at[pl.ds(base, _BW)], iidx_v)

    lanes = lax.iota(jnp.int32, _L)

    bias_cps = [
        pltpu.async_copy(ubias_hbm.at[uidx_v], ub_v, ubsem),
        pltpu.async_copy(ibias_hbm.at[iidx_v], ib_v, ibsem),
    ]
    for cp in bias_cps:
        cp.wait()

    def fire(pbase, buf):
        ue_d, ie_d = ue_v[buf], ie_v[buf]

        def fire16(s, carry2):
            jb = s * _L
            sl = pl.ds(pbase + jb, _L)
            u16 = uidx_v[sl]
            i16 = iidx_v[sl]
            for r in range(_L):
                u = u16[r]
                i = i16[r]
                dst = pl.ds(jb + r, 1)
                pltpu.async_copy(uemb_hbm.at[pl.ds(u, 1), :],
                                 ue_d.at[dst, :], usem[buf])
                pltpu.async_copy(iemb_hbm.at[pl.ds(i, 1), :],
                                 ie_d.at[dst, :], isem[buf])
            return carry2

        lax.fori_loop(0, _C // _L, fire16, 0)

    def drain(buf):
        pltpu.make_async_copy(uemb_hbm.at[pl.ds(0, _C), :], ue_v[buf], usem[buf]).wait()
        pltpu.make_async_copy(iemb_hbm.at[pl.ds(0, _C), :], ie_v[buf], isem[buf]).wait()

    def compute(pbase, buf):
        ue_d, ie_d = ue_v[buf], ie_v[buf]

        def group(g, carry2):
            for r in range(_L):
                row = g * _L + r
                acc = ue_d[row, pl.ds(0, _L)] * ie_d[row, pl.ds(0, _L)]
                for c in range(1, EMB // _L):
                    acc = acc + (ue_d[row, pl.ds(c * _L, _L)]
                                 * ie_d[row, pl.ds(c * _L, _L)])
                plsc.store_scatter(tmp_v, [lanes * _L + r], acc)
            tot = tmp_v[pl.ds(0, _L)]
            for j in range(1, _L):
                tot = tot + tmp_v[pl.ds(j * _L, _L)]
            osl = pl.ds(pbase + g * _L, _L)
            o_v[osl] = tot + ub_v[osl] + ib_v[osl]
            return carry2

        lax.fori_loop(0, _C // _L, group, 0)

    fire(0, 0)
    fire(_C, 1)
    drain(0)
    compute(0, 0)
    fire(2 * _C, 0)
    drain(1)
    compute(_C, 1)
    fire(3 * _C, 1)
    drain(0)
    compute(2 * _C, 0)
    drain(1)
    compute(3 * _C, 1)

    pltpu.sync_copy(o_v, out_hbm.at[pl.ds(base, _BW)])


@jax.jit
def _mf_call(user, item, uemb, iemb, ubias, ibias):
    mesh = plsc.VectorSubcoreMesh(core_axis_name="c", subcore_axis_name="s")
    f = functools.partial(
        pl.kernel,
        out_type=jax.ShapeDtypeStruct((B,), jnp.float32),
        mesh=mesh,
        compiler_params=pltpu.CompilerParams(
            needs_layout_passes=False,
            use_tc_tiling_on_sc=True,
        ),
        scratch_types=[
            pltpu.VMEM((_BW,), jnp.int32),
            pltpu.VMEM((_BW,), jnp.int32),
            [pltpu.VMEM((_C, EMB), jnp.float32) for _ in range(_NB)],
            [pltpu.VMEM((_C, EMB), jnp.float32) for _ in range(_NB)],
            pltpu.VMEM((_BW,), jnp.float32),
            pltpu.VMEM((_BW,), jnp.float32),
            pltpu.VMEM((_BW,), jnp.float32),
            pltpu.VMEM((_L * _L,), jnp.float32),
            [pltpu.SemaphoreType.DMA for _ in range(_NB)],
            [pltpu.SemaphoreType.DMA for _ in range(_NB)],
            pltpu.SemaphoreType.DMA,
            pltpu.SemaphoreType.DMA,
        ],
    )(_mf_body)
    return f(user, item, uemb, iemb, ubias, ibias)


def kernel(user, item, user_emb_w, item_emb_w, user_bias_w, item_bias_w):
    return _mf_call(
        user.astype(jnp.int32),
        item.astype(jnp.int32),
        user_emb_w,
        item_emb_w,
        user_bias_w.reshape(-1),
        item_bias_w.reshape(-1),
    )


# speedup vs baseline: 1.3358x; 1.0119x over previous
import functools

import jax
import jax.numpy as jnp
from jax import lax
from jax.experimental import pallas as pl
from jax.experimental.pallas import tpu as pltpu
from jax.experimental.pallas import tpu_sc as plsc

N_ROWS = 100000
EMB = 64
B = 16384

_NC = 2
_NS = 16
_NW = _NC * _NS
_BW = B // _NW
_L = 16
_C = 128
_NB = 2


def _mf_body(user_hbm, item_hbm, uemb_hbm, iemb_hbm, ubias_hbm, ibias_hbm,
             out_hbm, uidx_v, iidx_v, ue_v, ie_v, ub_v, ib_v, o_v, tmp_v,
             usem, isem, ubsem, ibsem):
    wid = lax.axis_index("s") * _NC + lax.axis_index("c")
    base = wid * _BW

    pltpu.sync_copy(user_hbm.at[pl.ds(base, _BW)], uidx_v)
    pltpu.sync_copy(item_hbm.at[pl.ds(base, _BW)], iidx_v)

    lanes = lax.iota(jnp.int32, _L)

    bias_cps = [
        pltpu.async_copy(ubias_hbm.at[uidx_v], ub_v, ubsem),
        pltpu.async_copy(ibias_hbm.at[iidx_v], ib_v, ibsem),
    ]

    def fire(pbase, buf):
        ue_d, ie_d = ue_v[buf], ie_v[buf]

        def fire16(s, carry2):
            jb = s * _L
            sl = pl.ds(pbase + jb, _L)
            u16 = uidx_v[sl]
            i16 = iidx_v[sl]
            for r in range(_L):
                u = u16[r]
                i = i16[r]
                dst = pl.ds(jb + r, 1)
                pltpu.async_copy(uemb_hbm.at[pl.ds(u, 1), :],
                                 ue_d.at[dst, :], usem[buf])
                pltpu.async_copy(iemb_hbm.at[pl.ds(i, 1), :],
                                 ie_d.at[dst, :], isem[buf])
            return carry2

        lax.fori_loop(0, _C // _L, fire16, 0)

    def drain(buf):
        pltpu.make_async_copy(uemb_hbm.at[pl.ds(0, _C), :], ue_v[buf], usem[buf]).wait()
        pltpu.make_async_copy(iemb_hbm.at[pl.ds(0, _C), :], ie_v[buf], isem[buf]).wait()

    def compute(pbase, buf):
        ue_d, ie_d = ue_v[buf], ie_v[buf]

        def group(g, carry2):
            for r in range(_L):
                row = g * _L + r
                acc = ue_d[row, pl.ds(0, _L)] * ie_d[row, pl.ds(0, _L)]
                for c in range(1, EMB // _L):
                    acc = acc + (ue_d[row, pl.ds(c * _L, _L)]
                                 * ie_d[row, pl.ds(c * _L, _L)])
                plsc.store_scatter(tmp_v, [lanes * _L + r], acc)
            tot = tmp_v[pl.ds(0, _L)]
            for j in range(1, _L):
                tot = tot + tmp_v[pl.ds(j * _L, _L)]
            osl = pl.ds(pbase + g * _L, _L)
            o_v[osl] = tot + ub_v[osl] + ib_v[osl]
            return carry2

        lax.fori_loop(0, _C // _L, group, 0)

    fire(0, 0)
    fire(_C, 1)
    for cp in bias_cps:
        cp.wait()
    drain(0)
    compute(0, 0)
    fire(2 * _C, 0)
    drain(1)
    compute(_C, 1)
    fire(3 * _C, 1)
    drain(0)
    compute(2 * _C, 0)
    drain(1)
    compute(3 * _C, 1)

    pltpu.sync_copy(o_v, out_hbm.at[pl.ds(base, _BW)])


@jax.jit
def _mf_call(user, item, uemb, iemb, ubias, ibias):
    mesh = plsc.VectorSubcoreMesh(core_axis_name="c", subcore_axis_name="s")
    f = functools.partial(
        pl.kernel,
        out_type=jax.ShapeDtypeStruct((B,), jnp.float32),
        mesh=mesh,
        compiler_params=pltpu.CompilerParams(
            needs_layout_passes=False,
            use_tc_tiling_on_sc=True,
        ),
        scratch_types=[
            pltpu.VMEM((_BW,), jnp.int32),
            pltpu.VMEM((_BW,), jnp.int32),
            [pltpu.VMEM((_C, EMB), jnp.float32) for _ in range(_NB)],
            [pltpu.VMEM((_C, EMB), jnp.float32) for _ in range(_NB)],
            pltpu.VMEM((_BW,), jnp.float32),
            pltpu.VMEM((_BW,), jnp.float32),
            pltpu.VMEM((_BW,), jnp.float32),
            pltpu.VMEM((_L * _L,), jnp.float32),
            [pltpu.SemaphoreType.DMA for _ in range(_NB)],
            [pltpu.SemaphoreType.DMA for _ in range(_NB)],
            pltpu.SemaphoreType.DMA,
            pltpu.SemaphoreType.DMA,
        ],
    )(_mf_body)
    return f(user, item, uemb, iemb, ubias, ibias)


def kernel(user, item, user_emb_w, item_emb_w, user_bias_w, item_bias_w):
    return _mf_call(
        user.astype(jnp.int32),
        item.astype(jnp.int32),
        user_emb_w,
        item_emb_w,
        user_bias_w.reshape(-1),
        item_bias_w.reshape(-1),
    )


# speedup vs baseline: 1.3393x; 1.0027x over previous
import functools

import jax
import jax.numpy as jnp
from jax import lax
from jax.experimental import pallas as pl
from jax.experimental.pallas import tpu as pltpu
from jax.experimental.pallas import tpu_sc as plsc

N_ROWS = 100000
EMB = 64
B = 16384

_NC = 2
_NS = 16
_NW = _NC * _NS
_BW = B // _NW
_L = 16
_C = 128
_NB = 2


def _mf_body(user_hbm, item_hbm, uemb_hbm, iemb_hbm, ubias_hbm, ibias_hbm,
             out_hbm, uidx_v, iidx_v, ue_v, ie_v, ub_v, ib_v, o_v, tmp_v,
             usem, isem, ubsem, ibsem):
    wid = lax.axis_index("s") * _NC + lax.axis_index("c")
    base = wid * _BW

    idx_cps = [
        pltpu.async_copy(user_hbm.at[pl.ds(base, _BW)], uidx_v, usem[0]),
        pltpu.async_copy(item_hbm.at[pl.ds(base, _BW)], iidx_v, isem[0]),
    ]
    for cp in idx_cps:
        cp.wait()

    lanes = lax.iota(jnp.int32, _L)

    bias_cps = [
        pltpu.async_copy(ubias_hbm.at[uidx_v], ub_v, ubsem),
        pltpu.async_copy(ibias_hbm.at[iidx_v], ib_v, ibsem),
    ]

    def fire(pbase, buf):
        ue_d, ie_d = ue_v[buf], ie_v[buf]

        def fire16(s, carry2):
            jb = s * _L
            sl = pl.ds(pbase + jb, _L)
            u16 = uidx_v[sl]
            i16 = iidx_v[sl]
            for r in range(_L):
                u = u16[r]
                i = i16[r]
                dst = pl.ds(jb + r, 1)
                pltpu.async_copy(uemb_hbm.at[pl.ds(u, 1), :],
                                 ue_d.at[dst, :], usem[buf])
                pltpu.async_copy(iemb_hbm.at[pl.ds(i, 1), :],
                                 ie_d.at[dst, :], isem[buf])
            return carry2

        lax.fori_loop(0, _C // _L, fire16, 0)

    def drain(buf):
        pltpu.make_async_copy(uemb_hbm.at[pl.ds(0, _C), :], ue_v[buf], usem[buf]).wait()
        pltpu.make_async_copy(iemb_hbm.at[pl.ds(0, _C), :], ie_v[buf], isem[buf]).wait()

    def compute(pbase, buf):
        ue_d, ie_d = ue_v[buf], ie_v[buf]

        def group(g, carry2):
            for r in range(_L):
                row = g * _L + r
                acc = ue_d[row, pl.ds(0, _L)] * ie_d[row, pl.ds(0, _L)]
                for c in range(1, EMB // _L):
                    acc = acc + (ue_d[row, pl.ds(c * _L, _L)]
                                 * ie_d[row, pl.ds(c * _L, _L)])
                plsc.store_scatter(tmp_v, [lanes * _L + r], acc)
            tot = tmp_v[pl.ds(0, _L)]
            for j in range(1, _L):
                tot = tot + tmp_v[pl.ds(j * _L, _L)]
            osl = pl.ds(pbase + g * _L, _L)
            o_v[osl] = tot + ub_v[osl] + ib_v[osl]
            return carry2

        lax.fori_loop(0, _C // _L, group, 0)

    fire(0, 0)
    fire(_C, 1)
    for cp in bias_cps:
        cp.wait()
    drain(0)
    compute(0, 0)
    fire(2 * _C, 0)
    drain(1)
    compute(_C, 1)
    fire(3 * _C, 1)
    drain(0)
    compute(2 * _C, 0)
    drain(1)
    compute(3 * _C, 1)

    pltpu.sync_copy(o_v, out_hbm.at[pl.ds(base, _BW)])


@jax.jit
def _mf_call(user, item, uemb, iemb, ubias, ibias):
    mesh = plsc.VectorSubcoreMesh(core_axis_name="c", subcore_axis_name="s")
    f = functools.partial(
        pl.kernel,
        out_type=jax.ShapeDtypeStruct((B,), jnp.float32),
        mesh=mesh,
        compiler_params=pltpu.CompilerParams(
            needs_layout_passes=False,
            use_tc_tiling_on_sc=True,
        ),
        scratch_types=[
            pltpu.VMEM((_BW,), jnp.int32),
            pltpu.VMEM((_BW,), jnp.int32),
            [pltpu.VMEM((_C, EMB), jnp.float32) for _ in range(_NB)],
            [pltpu.VMEM((_C, EMB), jnp.float32) for _ in range(_NB)],
            pltpu.VMEM((_BW,), jnp.float32),
            pltpu.VMEM((_BW,), jnp.float32),
            pltpu.VMEM((_BW,), jnp.float32),
            pltpu.VMEM((_L * _L,), jnp.float32),
            [pltpu.SemaphoreType.DMA for _ in range(_NB)],
            [pltpu.SemaphoreType.DMA for _ in range(_NB)],
            pltpu.SemaphoreType.DMA,
            pltpu.SemaphoreType.DMA,
        ],
    )(_mf_body)
    return f(user, item, uemb, iemb, ubias, ibias)


def kernel(user, item, user_emb_w, item_emb_w, user_bias_w, item_bias_w):
    return _mf_call(
        user.astype(jnp.int32),
        item.astype(jnp.int32),
        user_emb_w,
        item_emb_w,
        user_bias_w.reshape(-1),
        item_bias_w.reshape(-1),
    )
